# baseline (speedup 1.0000x reference)
import functools

import jax
import jax.numpy as jnp
from jax import lax
from jax.experimental import pallas as pl
from jax.experimental.pallas import tpu as pltpu

S = 2048
NH = 12
HD = 64
BLK = 32
SCB = S // BLK
WIN = 512
TOPK = 8
NEG = -1e30
QT = 256
HPP = 2


def _nsa_pair_kernel(q_ref, k_ref, v_ref, kc_ref, vc_ref, g_ref, o_ref,
                     qs_ref, ks_ref, vs_ref):
    scale = HD ** -0.5

    qb2 = q_ref[...].astype(jnp.bfloat16)
    kb2 = k_ref[...].astype(jnp.bfloat16)
    vb2 = v_ref[...].astype(jnp.bfloat16)

    tcol = lax.broadcasted_iota(jnp.int32, (SCB, S), 1)
    brow = lax.broadcasted_iota(jnp.int32, (SCB, S), 0)
    memb = (tcol // BLK == brow).astype(jnp.bfloat16)
    cmaskT = (brow + 1) * BLK - 1 <= tcol

    for t in range(HPP):
        lo, hi_ = t * HD, (t + 1) * HD
        qs_ref[...] = qb2[:, lo:hi_]
        ks_ref[...] = kb2[:, lo:hi_]
        vs_ref[...] = vb2[:, lo:hi_]
        qb = qs_ref[...]
        kb = ks_ref[...]
        vb = vs_ref[...]
        kc = kc_ref[:, lo:hi_]
        vc = vc_ref[:, lo:hi_]
        gw = jax.nn.sigmoid(g_ref[0, t, 0])
        gc = jax.nn.sigmoid(g_ref[1, t, 0])
        gs = jax.nn.sigmoid(g_ref[2, t, 0])

        vaug = jnp.concatenate(
            [vb, jnp.ones((S, 1), dtype=jnp.bfloat16)], axis=1)

        lcT = lax.dot_general(kc.astype(jnp.bfloat16), qb,
                              (((1,), (1,)), ((), ())),
                              preferred_element_type=jnp.float32) * scale
        lcTm = jnp.where(cmaskT, lcT, NEG)

        work = lcTm
        mC = jnp.max(work, axis=0, keepdims=True)
        m = mC
        for _ in range(TOPK - 1):
            work = jnp.where(work == m, NEG * 4.0, work)
            m = jnp.max(work, axis=0, keepdims=True)
        selT = jnp.where((lcTm >= m) & cmaskT, 1.0, 0.0).astype(jnp.bfloat16)

        pcT = jnp.exp(lcTm - mC)
        any_c = mC[0] > (NEG * 0.5)
        gcn = jnp.where(any_c, gc, 0.0) / jnp.sum(pcT, axis=0)
        o_cmp = lax.dot_general(pcT * gcn[None, :], vc,
                                (((0,), (0,)), ((), ())),
                                preferred_element_type=jnp.float32)

        for i in range(S // QT):
            qs = i * QT
            ke = (i + 1) * QT
            l = jnp.dot(qb[qs:qs + QT, :], kb[:ke, :].T,
                        preferred_element_type=jnp.float32) * scale

            sb = max(0, qs - WIN)
            rows = qs + lax.broadcasted_iota(jnp.int32, (QT, ke - sb), 0)
            cols = sb + lax.broadcasted_iota(jnp.int32, (QT, ke - sb), 1)
            win_b = jnp.where((rows - cols).astype(jnp.uint32) < WIN,
                              1.0, 0.0).astype(jnp.bfloat16)

            sel_tok = lax.dot_general(selT[:, qs:qs + QT], memb[:, :ke],
                                      (((0,), (0,)), ((), ())),
                                      preferred_element_type=jnp.float32)
            m = jnp.max(l, axis=-1, keepdims=True)
            eb = jnp.exp(l - m).astype(jnp.bfloat16)
            ewb = win_b * eb[:, sb:ke]
            esb = sel_tok.astype(jnp.bfloat16) * eb
            ow = jnp.dot(ewb, vaug[sb:ke, :], preferred_element_type=jnp.float32)
            os_ = jnp.dot(esb, vaug[:ke, :], preferred_element_type=jnp.float32)
            cw = gw[qs:qs + QT] / jnp.maximum(ow[:, HD], 1e-30)
            cs = gs[qs:qs + QT] / jnp.maximum(os_[:, HD], 1e-30)
            o_tile = (cw[:, None] * ow[:, :HD] + cs[:, None] * os_[:, :HD]
                      + o_cmp[qs:qs + QT, :])
            o_ref[qs:qs + QT, lo:hi_] = o_tile


@functools.partial(jax.jit, static_argnames=())
def kernel(q, k, v, g_win, g_cmp, g_slt):
    q2 = q.reshape(S, NH * HD)
    k2 = k.reshape(S, NH * HD)
    v2 = v.reshape(S, NH * HD)
    kc2 = k.reshape(SCB, BLK, NH * HD).mean(axis=1)
    vc2 = v.reshape(SCB, BLK, NH * HD).mean(axis=1)
    gall = jnp.transpose(jnp.stack([g_win, g_cmp, g_slt]), (0, 2, 1))
    gall = gall.reshape(3, NH, 1, S)

    lane = pl.BlockSpec((S, HPP * HD), lambda g: (0, g))
    lanec = pl.BlockSpec((SCB, HPP * HD), lambda g: (0, g))
    sg = pl.BlockSpec((3, HPP, 1, S), lambda g: (0, g, 0, 0))
    o = pl.pallas_call(
        _nsa_pair_kernel,
        grid=(NH // HPP,),
        in_specs=[lane, lane, lane, lanec, lanec, sg],
        out_specs=lane,
        out_shape=jax.ShapeDtypeStruct((S, NH * HD), jnp.float32),
        scratch_shapes=[pltpu.VMEM((S, HD), jnp.bfloat16)] * 3,
        compiler_params=pltpu.CompilerParams(
            dimension_semantics=("parallel",)),
    )(q2, k2, v2, kc2, vc2, gall)
    return o.reshape(S, NH, HD)


# speedup vs baseline: 1.2217x; 1.2217x over previous
import functools

import jax
import jax.numpy as jnp
from jax import lax
from jax.experimental import pallas as pl
from jax.experimental.pallas import tpu as pltpu

S = 2048
NH = 12
HD = 64
BLK = 32
SCB = S // BLK
WIN = 512
TOPK = 8
NEG = -1e30
QT = 256


def _nsa_head_kernel(q_ref, k_ref, v_ref, kc_ref, vc_ref, g_ref, o_ref):
    qh = q_ref[0]
    qb = qh.astype(jnp.bfloat16)
    kb = k_ref[0].astype(jnp.bfloat16)
    vb = v_ref[0].astype(jnp.bfloat16)
    kc = kc_ref[0]
    vc = vc_ref[0]
    gw = jax.nn.sigmoid(g_ref[0, 0, 0])
    gc = jax.nn.sigmoid(g_ref[1, 0, 0])
    gs = jax.nn.sigmoid(g_ref[2, 0, 0])
    scale = HD ** -0.5

    vaug = jnp.concatenate(
        [vb, jnp.ones((S, 1), dtype=jnp.bfloat16)], axis=1)

    tcol = lax.broadcasted_iota(jnp.int32, (SCB, S), 1)
    brow = lax.broadcasted_iota(jnp.int32, (SCB, S), 0)
    memb = (tcol // BLK == brow).astype(jnp.bfloat16)

    lcT = lax.dot_general(kc.astype(jnp.bfloat16), qb,
                          (((1,), (1,)), ((), ())),
                          preferred_element_type=jnp.float32) * scale
    cmaskT = (brow + 1) * BLK - 1 <= tcol
    lcTm = jnp.where(cmaskT, lcT, NEG)

    work = lcTm
    mC = jnp.max(work, axis=0, keepdims=True)
    m = mC
    for _ in range(TOPK - 1):
        work = jnp.where(work == m, NEG * 4.0, work)
        m = jnp.max(work, axis=0, keepdims=True)
    selT = jnp.where((lcTm >= m) & cmaskT, 1.0, 0.0).astype(jnp.bfloat16)

    pcT = jnp.exp(lcTm - mC)
    any_c = mC[0] > (NEG * 0.5)
    gcn = jnp.where(any_c, gc, 0.0) / jnp.sum(pcT, axis=0)
    o_cmp = lax.dot_general(pcT * gcn[None, :], vc,
                            (((0,), (0,)), ((), ())),
                            preferred_element_type=jnp.float32)

    for i in range(S // QT):
        qs = i * QT
        ke = (i + 1) * QT
        l = jnp.dot(qb[qs:qs + QT, :], kb[:ke, :].T,
                    preferred_element_type=jnp.float32) * scale

        sb = max(0, qs - WIN)
        W = ke - sb
        rows = qs + lax.broadcasted_iota(jnp.int32, (QT, W), 0)
        cols = sb + lax.broadcasted_iota(jnp.int32, (QT, W), 1)
        win_b = jnp.where((rows - cols).astype(jnp.uint32) < WIN,
                          1.0, 0.0).astype(jnp.bfloat16)

        sel_tok = lax.dot_general(selT[:, qs:qs + QT], memb[:, :ke],
                                  (((0,), (0,)), ((), ())),
                                  preferred_element_type=jnp.float32)
        m = jnp.max(l, axis=-1, keepdims=True)
        eb = jnp.exp(l - m).astype(jnp.bfloat16)
        ewb = win_b * eb[:, sb:ke]
        esb = sel_tok.astype(jnp.bfloat16) * eb
        ow = jnp.dot(ewb, vaug[sb:ke, :], preferred_element_type=jnp.float32)
        os_ = jnp.dot(esb, vaug[:ke, :], preferred_element_type=jnp.float32)
        cw = gw[qs:qs + QT] / jnp.maximum(ow[:, HD], 1e-30)
        cs = gs[qs:qs + QT] / jnp.maximum(os_[:, HD], 1e-30)
        o_tile = (cw[:, None] * ow[:, :HD] + cs[:, None] * os_[:, :HD]
                  + o_cmp[qs:qs + QT, :])
        o_ref[0, qs:qs + QT, :] = o_tile


@functools.partial(jax.jit, static_argnames=())
def kernel(q, k, v, g_win, g_cmp, g_slt):
    qh = jnp.transpose(q, (1, 0, 2))
    kh = jnp.transpose(k, (1, 0, 2))
    vh = jnp.transpose(v, (1, 0, 2))
    kc = kh.reshape(NH, SCB, BLK, HD).mean(axis=2)
    vc = vh.reshape(NH, SCB, BLK, HD).mean(axis=2)
    gall = jnp.transpose(jnp.stack([g_win, g_cmp, g_slt]), (0, 2, 1))
    gall = gall.reshape(3, NH, 1, S)

    shd = pl.BlockSpec((1, S, HD), lambda h: (h, 0, 0))
    sc = pl.BlockSpec((1, SCB, HD), lambda h: (h, 0, 0))
    sg = pl.BlockSpec((3, 1, 1, S), lambda h: (0, h, 0, 0))
    o = pl.pallas_call(
        _nsa_head_kernel,
        grid=(NH,),
        in_specs=[shd, shd, shd, sc, sc, sg],
        out_specs=shd,
        out_shape=jax.ShapeDtypeStruct((NH, S, HD), jnp.float32),
        compiler_params=pltpu.CompilerParams(
            dimension_semantics=("parallel",)),
    )(qh, kh, vh, kc, vc, gall)
    return jnp.transpose(o, (1, 0, 2))
